# baseline (speedup 1.0000x reference)
import functools

import jax
import jax.numpy as jnp
from jax import lax
from jax.experimental import pallas as pl
from jax.experimental.pallas import tpu as pltpu
from jax.experimental.pallas import tpu_sc as plsc

_HALO = 4
_LANES = 16


def _sc_tok_len(ids, T):
    NC, NS = 2, 16
    NW = NC * NS
    chunk = T // NW
    assert chunk * NW == T and chunk % _LANES == 0

    mesh = plsc.VectorSubcoreMesh(core_axis_name="c", subcore_axis_name="s")
    PAD = 8

    @functools.partial(
        pl.kernel,
        mesh=mesh,
        out_type=jax.ShapeDtypeStruct((T,), jnp.int32),
        scratch_types=[
            pltpu.VMEM((chunk + 2 * PAD,), jnp.int32),
            pltpu.VMEM((chunk,), jnp.int32),
            pltpu.SemaphoreType.DMA,
        ],
    )
    def sc_body(ids_hbm, out_hbm, ids_v, tl_v, sem):
        wid = lax.axis_index("s") * NC + lax.axis_index("c")
        base = pl.multiple_of(wid * chunk, 8)
        lo = pl.multiple_of(jnp.maximum(base - PAD, 0), 8)
        hi = pl.multiple_of(jnp.minimum(base + chunk, T - PAD), 8)
        cp0 = pltpu.async_copy(ids_hbm.at[pl.ds(base, chunk)],
                               ids_v.at[pl.ds(PAD, chunk)], sem)
        cp1 = pltpu.async_copy(ids_hbm.at[pl.ds(lo, PAD)],
                               ids_v.at[pl.ds(0, PAD)], sem)
        cp2 = pltpu.async_copy(ids_hbm.at[pl.ds(hi, PAD)],
                               ids_v.at[pl.ds(PAD + chunk, PAD)], sem)
        cp0.wait()
        cp1.wait()
        cp2.wait()
        lane = lax.iota(jnp.int32, _LANES)
        sent = jnp.full((_LANES,), -1, jnp.int32)

        @pl.when(wid == 0)
        def _():
            v = ids_v[pl.ds(0, _LANES)]
            ids_v[pl.ds(0, _LANES)] = jnp.where(lane < PAD, sent, v)

        @pl.when(wid == NW - 1)
        def _():
            v = ids_v[pl.ds(chunk + 2 * PAD - _LANES, _LANES)]
            ids_v[pl.ds(chunk + 2 * PAD - _LANES, _LANES)] = jnp.where(
                lane >= _LANES - PAD, sent, v)

        one = jnp.ones((_LANES,), jnp.int32)
        zero = jnp.zeros((_LANES,), jnp.int32)

        def step(i, carry):
            b = i * _LANES
            c = ids_v[pl.ds(b + PAD, _LANES)]
            cnt = one
            for k in range(2 * _HALO + 1):
                if k == _HALO:
                    continue
                n = ids_v[pl.ds(b + PAD - _HALO + k, _LANES)]
                cnt = cnt + jnp.where(n == c, one, zero)
            tl_v[pl.ds(b, _LANES)] = jnp.minimum(cnt, _HALO + 1) - 1
            return carry

        lax.fori_loop(0, chunk // _LANES, step, 0, unroll=4)
        pltpu.sync_copy(tl_v, out_hbm.at[pl.ds(base, chunk)])

    return sc_body(ids)


def _tc_apply(flat, grid, tok_len, cu_heads):
    T, D_A = flat.shape
    S, _, D_B = grid.shape
    BT = 8192
    n_heads = cu_heads.shape[0]
    gstack = grid.reshape(S * D_A, D_B)

    def body(cu_ref, tl_ref, flat_ref, g_ref, out_ref):
        x = flat_ref[...]
        tl = tl_ref[...].reshape(1, BT)
        row = pl.program_id(0) * BT + lax.broadcasted_iota(jnp.int32, (1, BT), 1)
        is_bos = row == cu_ref[0]
        for j in range(1, n_heads):
            is_bos = is_bos | (row == cu_ref[j])
        tl = jnp.where(is_bos, -1, tl)
        tlc = tl.reshape(BT, 1)
        xp = jnp.concatenate(
            [jnp.where(tlc == s, x, 0.0) for s in range(S)], axis=1)
        out_ref[...] = jnp.dot(xp, g_ref[...], preferred_element_type=jnp.float32)

    return pl.pallas_call(
        body,
        grid=(T // BT,),
        in_specs=[
            pl.BlockSpec(memory_space=pltpu.MemorySpace.SMEM),
            pl.BlockSpec((BT,), lambda i: (i,)),
            pl.BlockSpec((BT, D_A), lambda i: (i, 0)),
            pl.BlockSpec((S * D_A, D_B), lambda i: (0, 0)),
        ],
        out_specs=pl.BlockSpec((BT, D_B), lambda i: (i, 0)),
        out_shape=jax.ShapeDtypeStruct((T, D_B), jnp.float32),
    )(cu_heads, tok_len, flat, gstack)


@jax.jit
def kernel(flat, grid, segment_ids, cu_seqlens):
    T = flat.shape[0]
    ids = segment_ids.astype(jnp.int32)
    cu_heads = cu_seqlens[:-1].astype(jnp.int32)
    tok_len = _sc_tok_len(ids, T)
    return _tc_apply(flat, grid, tok_len, cu_heads)


# speedup vs baseline: 1.0154x; 1.0154x over previous
import functools

import jax
import jax.numpy as jnp
from jax import lax
from jax.experimental import pallas as pl
from jax.experimental.pallas import tpu as pltpu
from jax.experimental.pallas import tpu_sc as plsc

_HALO = 4
_LANES = 16


def _sc_tok_len(ids, T):
    NC, NS = 2, 16
    NW = NC * NS
    chunk = T // NW
    assert chunk * NW == T and chunk % _LANES == 0

    mesh = plsc.VectorSubcoreMesh(core_axis_name="c", subcore_axis_name="s")
    PAD = 8

    @functools.partial(
        pl.kernel,
        mesh=mesh,
        out_type=jax.ShapeDtypeStruct((T,), jnp.int32),
        scratch_types=[
            pltpu.VMEM((chunk + 2 * PAD,), jnp.int32),
            pltpu.VMEM((chunk,), jnp.int32),
            pltpu.SemaphoreType.DMA,
        ],
    )
    def sc_body(ids_hbm, out_hbm, ids_v, tl_v, sem):
        wid = lax.axis_index("s") * NC + lax.axis_index("c")
        base = pl.multiple_of(wid * chunk, 8)
        lo = pl.multiple_of(jnp.maximum(base - PAD, 0), 8)
        hi = pl.multiple_of(jnp.minimum(base + chunk, T - PAD), 8)
        cp0 = pltpu.async_copy(ids_hbm.at[pl.ds(base, chunk)],
                               ids_v.at[pl.ds(PAD, chunk)], sem)
        cp1 = pltpu.async_copy(ids_hbm.at[pl.ds(lo, PAD)],
                               ids_v.at[pl.ds(0, PAD)], sem)
        cp2 = pltpu.async_copy(ids_hbm.at[pl.ds(hi, PAD)],
                               ids_v.at[pl.ds(PAD + chunk, PAD)], sem)
        cp0.wait()
        cp1.wait()
        cp2.wait()
        lane = lax.iota(jnp.int32, _LANES)
        sent = jnp.full((_LANES,), -1, jnp.int32)

        @pl.when(wid == 0)
        def _():
            v = ids_v[pl.ds(0, _LANES)]
            ids_v[pl.ds(0, _LANES)] = jnp.where(lane < PAD, sent, v)

        @pl.when(wid == NW - 1)
        def _():
            v = ids_v[pl.ds(chunk + 2 * PAD - _LANES, _LANES)]
            ids_v[pl.ds(chunk + 2 * PAD - _LANES, _LANES)] = jnp.where(
                lane >= _LANES - PAD, sent, v)

        one = jnp.ones((_LANES,), jnp.int32)
        zero = jnp.zeros((_LANES,), jnp.int32)

        def step(i, carry):
            b = i * _LANES
            c = ids_v[pl.ds(b + PAD, _LANES)]
            cnt = one
            for k in range(2 * _HALO + 1):
                if k == _HALO:
                    continue
                n = ids_v[pl.ds(b + PAD - _HALO + k, _LANES)]
                cnt = cnt + jnp.where(n == c, one, zero)
            tl_v[pl.ds(b, _LANES)] = jnp.minimum(cnt, _HALO + 1) - 1
            return carry

        lax.fori_loop(0, chunk // _LANES, step, 0)
        pltpu.sync_copy(tl_v, out_hbm.at[pl.ds(base, chunk)])

    return sc_body(ids)


def _tc_apply(flat, grid, tok_len, cu_heads):
    T, D_A = flat.shape
    S, _, D_B = grid.shape
    BT = 8192
    n_heads = cu_heads.shape[0]
    gstack = grid.reshape(S * D_A, D_B)

    def body(cu_ref, tl_ref, flat_ref, g_ref, out_ref):
        x = flat_ref[...]
        tl = tl_ref[...].reshape(1, BT)
        row = pl.program_id(0) * BT + lax.broadcasted_iota(jnp.int32, (1, BT), 1)
        is_bos = row == cu_ref[0]
        for j in range(1, n_heads):
            is_bos = is_bos | (row == cu_ref[j])
        tl = jnp.where(is_bos, -1, tl)
        tlc = tl.reshape(BT, 1)
        xp = jnp.concatenate(
            [jnp.where(tlc == s, x, 0.0) for s in range(S)], axis=1)
        out_ref[...] = jnp.dot(xp, g_ref[...], preferred_element_type=jnp.float32)

    return pl.pallas_call(
        body,
        grid=(T // BT,),
        in_specs=[
            pl.BlockSpec(memory_space=pltpu.MemorySpace.SMEM),
            pl.BlockSpec((BT,), lambda i: (i,)),
            pl.BlockSpec((BT, D_A), lambda i: (i, 0)),
            pl.BlockSpec((S * D_A, D_B), lambda i: (0, 0)),
        ],
        out_specs=pl.BlockSpec((BT, D_B), lambda i: (i, 0)),
        out_shape=jax.ShapeDtypeStruct((T, D_B), jnp.float32),
    )(cu_heads, tok_len, flat, gstack)


@jax.jit
def kernel(flat, grid, segment_ids, cu_seqlens):
    T = flat.shape[0]
    ids = segment_ids.astype(jnp.int32)
    cu_heads = cu_seqlens[:-1].astype(jnp.int32)
    tok_len = _sc_tok_len(ids, T)
    return _tc_apply(flat, grid, tok_len, cu_heads)
